# baseline (speedup 1.0000x reference)
import functools

import jax
import jax.numpy as jnp
from jax import lax
from jax.experimental import pallas as pl
from jax.experimental.pallas import tpu as pltpu
from jax.experimental.pallas import tpu_sc as plsc

NC = 2
NS = 16
L = 16

B = 4
T = 4096
D = 2048
K = T // 2
RADIX = 256
PASSES = 4
CHUNK = T // L

RPC = 2 * K
RPW = RPC // NS
GCH = 8
NCH = RPW // GCH


def _digit(k_i32, shift):
    ku = plsc.bitcast(k_i32, jnp.uint32)
    du = jnp.bitwise_and(jnp.right_shift(ku, jnp.uint32(shift)), jnp.uint32(RADIX - 1))
    return plsc.bitcast(du, jnp.int32)


def _desc_key(bits_i32):
    sign = jnp.right_shift(bits_i32, 31)
    mask = jnp.bitwise_and(jnp.bitwise_not(sign), jnp.int32(0x7FFFFFFF))
    return jnp.bitwise_xor(bits_i32, mask)


UNROLL = 4


def _body(scores_hbm, hid_hbm, sel_hbm, bidx_hbm, idx_hbm, vals_hbm,
          sc_v, val_a, val_b, hist, rowbase,
          vstage, gstage, idx_v, buf0, buf1, sh_grow, sem0, sem1):
    c = lax.axis_index("c")
    s = lax.axis_index("s")

    @pl.when(s < 2)
    def _():
        b = c * 2 + s
        pltpu.sync_copy(scores_hbm.at[pl.ds(b * T, T)], sc_v)

        lane = lax.iota(jnp.int32, L)
        lane_c = lane * CHUNK
        ones = jnp.broadcast_to(jnp.int32(1), (L,))
        fifteen = jnp.broadcast_to(jnp.int32(15), (L,))
        lane_m1 = jnp.maximum(lane - 1, 0)
        zero_v = jnp.broadcast_to(jnp.int32(0), (L,))

        def key_of(v):
            x = plsc.load_gather(sc_v, [v])
            return _desc_key(plsc.bitcast(x, jnp.int32))

        srcs = [None, val_b, val_a, val_b]
        dsts = [val_b, val_a, val_b, val_a]
        for p in range(PASSES):
            shift = 8 * p
            src_v = srcs[p]
            dst_v = dsts[p]

            def zero_body(j, _):
                for u in range(UNROLL):
                    hist[j * UNROLL + u, :] = zero_v
                return 0
            lax.fori_loop(0, RADIX // UNROLL, zero_body, 0)

            def hist_body(i, _):
                for u in range(UNROLL):
                    idxv = lane_c + (i * UNROLL + u)
                    v = idxv if src_v is None else plsc.load_gather(src_v, [idxv])
                    d = _digit(key_of(v), shift)
                    plsc.addupdate_scatter(hist, [d, lane], ones)
                return 0
            lax.fori_loop(0, CHUNK // UNROLL, hist_body, 0)

            def cs_body(j, _):
                for u in range(UNROLL):
                    jj = j * UNROLL + u
                    hist[jj, :] = plsc.cumsum(hist[jj, :])
                return 0
            lax.fori_loop(0, RADIX // UNROLL, cs_body, 0)

            def rb_body(jj, carry):
                rows_vec = lane + jj * L
                tot = plsc.load_gather(hist, [rows_vec, fifteen])
                cs2 = plsc.cumsum(tot)
                plsc.store_scatter(rowbase, [rows_vec], cs2 - tot + carry)
                return carry + jnp.sum(tot)
            lax.fori_loop(0, RADIX // L, rb_body, jnp.int32(0))

            def fin_body(j, _):
                for u in range(UNROLL):
                    jj = j * UNROLL + u
                    jb = jnp.broadcast_to(jj, (L,))
                    shifted = plsc.load_gather(hist, [jb, lane_m1])
                    rb = plsc.load_gather(rowbase, [jb])
                    hist[jj, :] = jnp.where(lane > 0, shifted, 0) + rb
                return 0
            lax.fori_loop(0, RADIX // UNROLL, fin_body, 0)

            def perm_body(i, _):
                for u in range(UNROLL):
                    idxv = lane_c + (i * UNROLL + u)
                    v = idxv if src_v is None else plsc.load_gather(src_v, [idxv])
                    d = _digit(key_of(v), shift)
                    ofs = plsc.load_gather(hist, [d, lane])
                    plsc.store_scatter(dst_v, [ofs], v)
                    plsc.addupdate_scatter(hist, [d, lane], ones)
                return 0
            lax.fori_loop(0, CHUNK // UNROLL, perm_body, 0)

        def out_body(i, _):
            for u in range(UNROLL):
                sl = pl.ds((i * UNROLL + u) * L, L)
                v = val_a[sl]
                vstage[sl] = plsc.load_gather(sc_v, [v])
                gstage[sl] = v + b * T
            return 0
        lax.fori_loop(0, K // L // UNROLL, out_body, 0)

        pltpu.sync_copy(vstage, vals_hbm.at[pl.ds(b * K, K)])
        pltpu.sync_copy(val_a.at[pl.ds(0, K)], idx_hbm.at[pl.ds(b * K, K)])
        pltpu.sync_copy(gstage, sh_grow.at[pl.ds(s * K, K)])

        def bidx_body(i, _):
            for u in range(UNROLL):
                gstage[pl.ds((i * UNROLL + u) * L, L)] = jnp.broadcast_to(b, (L,))
            return 0
        lax.fori_loop(0, K // L // UNROLL, bidx_body, 0)
        pltpu.sync_copy(gstage, bidx_hbm.at[pl.ds(b * K, K)])

    plsc.subcore_barrier()

    pltpu.sync_copy(sh_grow.at[pl.ds(s * RPW, RPW)], idx_v)
    gbase = c * RPC + s * RPW

    bufs2 = (buf0, buf1)
    sems2 = (sem0, sem1)

    def start(ch):
        return pltpu.async_copy(
            hid_hbm.at[idx_v.at[pl.ds(ch * GCH, GCH)]], bufs2[ch % 2], sems2[ch % 2])

    pending = start(0)
    for ch in range(NCH):
        nxt = start(ch + 1) if ch + 1 < NCH else None
        pending.wait()
        pltpu.sync_copy(bufs2[ch % 2], sel_hbm.at[pl.ds(gbase + ch * GCH, GCH)])
        pending = nxt


_fused_call = functools.partial(
    pl.kernel,
    out_type=(
        jax.ShapeDtypeStruct((B * K, D), jnp.float32),
        jax.ShapeDtypeStruct((B * K,), jnp.int32),
        jax.ShapeDtypeStruct((B * K,), jnp.int32),
        jax.ShapeDtypeStruct((B * K,), jnp.float32),
    ),
    mesh=plsc.VectorSubcoreMesh(core_axis_name="c", subcore_axis_name="s"),
    compiler_params=pltpu.CompilerParams(needs_layout_passes=False),
    scratch_types=[
        pltpu.VMEM((T,), jnp.float32),
        pltpu.VMEM((T,), jnp.int32),
        pltpu.VMEM((T,), jnp.int32),
        pltpu.VMEM((RADIX, L), jnp.int32),
        pltpu.VMEM((RADIX,), jnp.int32),
        pltpu.VMEM((K,), jnp.float32),
        pltpu.VMEM((K,), jnp.int32),
        pltpu.VMEM((RPW,), jnp.int32),
        pltpu.VMEM((GCH, D), jnp.float32),
        pltpu.VMEM((GCH, D), jnp.float32),
        pltpu.VMEM_SHARED((RPC,), jnp.int32),
        pltpu.SemaphoreType.DMA,
        pltpu.SemaphoreType.DMA,
    ],
)(_body)


def kernel(scores, hidden_states):
    b, t, d = hidden_states.shape
    sel, bidx, idx, vals = _fused_call(
        scores.reshape(-1), hidden_states.reshape(b * t, d))
    return sel, bidx, idx, vals


# speedup vs baseline: 1.0001x; 1.0001x over previous
import functools

import jax
import jax.numpy as jnp
from jax import lax
from jax.experimental import pallas as pl
from jax.experimental.pallas import tpu as pltpu
from jax.experimental.pallas import tpu_sc as plsc

NC = 2
NS = 16
L = 16

B = 4
T = 4096
D = 2048
K = T // 2
RADIX = 256
PASSES = 4
CHUNK = T // L

RPC = 2 * K
RPW = RPC // NS
GCH = 8
NCH = RPW // GCH


def _digit(k_i32, shift):
    ku = plsc.bitcast(k_i32, jnp.uint32)
    du = jnp.bitwise_and(jnp.right_shift(ku, jnp.uint32(shift)), jnp.uint32(RADIX - 1))
    return plsc.bitcast(du, jnp.int32)


def _desc_key(bits_i32):
    sign = jnp.right_shift(bits_i32, 31)
    mask = jnp.bitwise_and(jnp.bitwise_not(sign), jnp.int32(0x7FFFFFFF))
    return jnp.bitwise_xor(bits_i32, mask)


UNROLL = 4


def _body(scores_hbm, hid_hbm, sel_hbm, bidx_hbm, idx_hbm, vals_hbm,
          sc_v, val_a, val_b, hist0, hist1, rowbase,
          vstage, gstage, idx0, idx1, buf0, buf1, sh_grow, sem0, sem1):
    c = lax.axis_index("c")
    s = lax.axis_index("s")

    @pl.when(s < 2)
    def _():
        b = c * 2 + s
        pltpu.sync_copy(scores_hbm.at[pl.ds(b * T, T)], sc_v)

        lane = lax.iota(jnp.int32, L)
        lane_c = lane * CHUNK
        ones = jnp.broadcast_to(jnp.int32(1), (L,))
        fifteen = jnp.broadcast_to(jnp.int32(15), (L,))
        lane_m1 = jnp.maximum(lane - 1, 0)
        zero_v = jnp.broadcast_to(jnp.int32(0), (L,))

        def premap_body(i, _):
            for u in range(UNROLL):
                sl = pl.ds((i * UNROLL + u) * L, L)
                bits = plsc.bitcast(sc_v[sl], jnp.int32)
                sc_v[sl] = plsc.bitcast(_desc_key(bits), jnp.float32)
            return 0
        lax.fori_loop(0, T // L // UNROLL, premap_body, 0)

        def key_of(v):
            return plsc.bitcast(plsc.load_gather(sc_v, [v]), jnp.int32)

        srcs = [None, val_b, val_a, val_b]
        dsts = [val_b, val_a, val_b, val_a]
        hq = (hist0, hist1)
        NQ = 2
        SUB = CHUNK // NQ
        for p in range(PASSES):
            shift = 8 * p
            src_v = srcs[p]
            dst_v = dsts[p]

            def zero_body(j, _):
                for u in range(UNROLL):
                    for q in range(NQ):
                        hq[q][j * UNROLL + u, :] = zero_v
                return 0
            lax.fori_loop(0, RADIX // UNROLL, zero_body, 0)

            def hist_body(i, _):
                for q in range(NQ):
                    idxv = lane_c + (q * SUB + i)
                    v = idxv if src_v is None else plsc.load_gather(src_v, [idxv])
                    d = _digit(key_of(v), shift)
                    plsc.addupdate_scatter(hq[q], [d, lane], ones)
                return 0
            lax.fori_loop(0, SUB, hist_body, 0)

            def cs_body(j, _):
                for u in range(2):
                    jj = j * 2 + u
                    t = hq[0][jj, :] + hq[1][jj, :]
                    hq[1][jj, :] = plsc.cumsum(t)
                return 0
            lax.fori_loop(0, RADIX // 2, cs_body, 0)

            def rb_body(jj, carry):
                rows_vec = lane + jj * L
                tot = plsc.load_gather(hq[1], [rows_vec, fifteen])
                cs2 = plsc.cumsum(tot)
                plsc.store_scatter(rowbase, [rows_vec], cs2 - tot + carry)
                return carry + jnp.sum(tot)
            lax.fori_loop(0, RADIX // L, rb_body, jnp.int32(0))

            def fin_body(j, _):
                for u in range(2):
                    jj = j * 2 + u
                    jb = jnp.broadcast_to(jj, (L,))
                    c0 = hq[0][jj, :]
                    incl = hq[1][jj, :]
                    shifted = plsc.load_gather(hq[1], [jb, lane_m1])
                    excl = jnp.where(lane > 0, shifted, 0)
                    rb = plsc.load_gather(rowbase, [jb])
                    base = excl + rb
                    hq[0][jj, :] = base
                    hq[1][jj, :] = base + c0
                return 0
            lax.fori_loop(0, RADIX // 2, fin_body, 0)

            def perm_body(i, _):
                for q in range(NQ):
                    idxv = lane_c + (q * SUB + i)
                    v = idxv if src_v is None else plsc.load_gather(src_v, [idxv])
                    d = _digit(key_of(v), shift)
                    ofs = plsc.load_gather(hq[q], [d, lane])
                    plsc.store_scatter(dst_v, [ofs], v)
                    plsc.addupdate_scatter(hq[q], [d, lane], ones)
                return 0
            lax.fori_loop(0, SUB, perm_body, 0)

        def out_body(i, _):
            for u in range(UNROLL):
                sl = pl.ds((i * UNROLL + u) * L, L)
                v = val_a[sl]
                k = plsc.bitcast(plsc.load_gather(sc_v, [v]), jnp.int32)
                vstage[sl] = plsc.bitcast(_desc_key(k), jnp.float32)
                gstage[sl] = v + b * T
            return 0
        lax.fori_loop(0, K // L // UNROLL, out_body, 0)

        pltpu.sync_copy(vstage, vals_hbm.at[pl.ds(b * K, K)])
        pltpu.sync_copy(val_a.at[pl.ds(0, K)], idx_hbm.at[pl.ds(b * K, K)])
        pltpu.sync_copy(gstage, sh_grow.at[pl.ds(s * K, K)])

        def bidx_body(i, _):
            for u in range(UNROLL):
                gstage[pl.ds((i * UNROLL + u) * L, L)] = jnp.broadcast_to(b, (L,))
            return 0
        lax.fori_loop(0, K // L // UNROLL, bidx_body, 0)
        pltpu.sync_copy(gstage, bidx_hbm.at[pl.ds(b * K, K)])

    plsc.subcore_barrier()

    gbase = c * RPC + s * RPW

    bufs2 = (buf0, buf1)
    sems2 = (sem0, sem1)
    idxb = (idx0, idx1)

    def start(ch):
        pltpu.sync_copy(sh_grow.at[pl.ds(s * RPW + ch * GCH, GCH)], idxb[ch % 2])
        return pltpu.async_copy(
            hid_hbm.at[idxb[ch % 2]], bufs2[ch % 2], sems2[ch % 2])

    pending = start(0)
    for ch in range(NCH):
        nxt = start(ch + 1) if ch + 1 < NCH else None
        pending.wait()
        pltpu.sync_copy(bufs2[ch % 2], sel_hbm.at[pl.ds(gbase + ch * GCH, GCH)])
        pending = nxt


_fused_call = functools.partial(
    pl.kernel,
    out_type=(
        jax.ShapeDtypeStruct((B * K, D), jnp.float32),
        jax.ShapeDtypeStruct((B * K,), jnp.int32),
        jax.ShapeDtypeStruct((B * K,), jnp.int32),
        jax.ShapeDtypeStruct((B * K,), jnp.float32),
    ),
    mesh=plsc.VectorSubcoreMesh(core_axis_name="c", subcore_axis_name="s"),
    compiler_params=pltpu.CompilerParams(needs_layout_passes=False),
    scratch_types=[
        pltpu.VMEM((T,), jnp.float32),
        pltpu.VMEM((T,), jnp.int32),
        pltpu.VMEM((T,), jnp.int32),
        pltpu.VMEM((RADIX, L), jnp.int32),
        pltpu.VMEM((RADIX, L), jnp.int32),
        pltpu.VMEM((RADIX,), jnp.int32),
        pltpu.VMEM((K,), jnp.float32),
        pltpu.VMEM((K,), jnp.int32),
        pltpu.VMEM((GCH,), jnp.int32),
        pltpu.VMEM((GCH,), jnp.int32),
        pltpu.VMEM((GCH, D), jnp.float32),
        pltpu.VMEM((GCH, D), jnp.float32),
        pltpu.VMEM_SHARED((RPC,), jnp.int32),
        pltpu.SemaphoreType.DMA,
        pltpu.SemaphoreType.DMA,
    ],
)(_body)


def kernel(scores, hidden_states):
    b, t, d = hidden_states.shape
    sel, bidx, idx, vals = _fused_call(
        scores.reshape(-1), hidden_states.reshape(b * t, d))
    return sel, bidx, idx, vals


# speedup vs baseline: 1.0982x; 1.0981x over previous
import functools

import jax
import jax.numpy as jnp
from jax import lax
from jax.experimental import pallas as pl
from jax.experimental.pallas import tpu as pltpu
from jax.experimental.pallas import tpu_sc as plsc

NC = 2
NS = 16
L = 16

B = 4
T = 4096
D = 2048
K = T // 2
RADIX = 256
PASSES = 4
CHUNK = T // L

RPC = 2 * K
RPW = RPC // NS
GCH = 8
NCH = RPW // GCH


def _digit(k_i32, shift):
    ku = plsc.bitcast(k_i32, jnp.uint32)
    du = jnp.bitwise_and(jnp.right_shift(ku, jnp.uint32(shift)), jnp.uint32(RADIX - 1))
    return plsc.bitcast(du, jnp.int32)


def _desc_key(bits_i32):
    sign = jnp.right_shift(bits_i32, 31)
    mask = jnp.bitwise_and(jnp.bitwise_not(sign), jnp.int32(0x7FFFFFFF))
    return jnp.bitwise_xor(bits_i32, mask)


UNROLL = 4


def _body(scores_hbm, hid_hbm, sel_hbm, bidx_hbm, idx_hbm, vals_hbm,
          sc_v, val_a, val_b, hist0, hist1, rowbase,
          vstage, gstage, idx0, idx1, buf0, buf1, sh_grow, sem0, sem1):
    c = lax.axis_index("c")
    s = lax.axis_index("s")

    @pl.when(s < 2)
    def _():
        b = c * 2 + s
        pltpu.sync_copy(scores_hbm.at[pl.ds(b * T, T)], sc_v.at[pl.ds(0, T)])

        lane = lax.iota(jnp.int32, L)
        lane_c = lane * CHUNK
        ones = jnp.broadcast_to(jnp.int32(1), (L,))
        fifteen = jnp.broadcast_to(jnp.int32(15), (L,))
        lane_m1 = jnp.maximum(lane - 1, 0)
        zero_v = jnp.broadcast_to(jnp.int32(0), (L,))

        def phys(e):
            return e + jnp.right_shift(e, 8)

        def premap_body(i, _):
            j = T // L // UNROLL - 1 - i
            for u in reversed(range(UNROLL)):
                e0 = (j * UNROLL + u) * L
                sl = pl.ds(e0, L)
                bits = plsc.bitcast(sc_v[sl], jnp.int32)
                key = plsc.bitcast(_desc_key(bits), jnp.float32)
                plsc.store_scatter(sc_v, [phys(e0 + lane)], key)
            return 0
        lax.fori_loop(0, T // L // UNROLL, premap_body, 0)

        def key_of(v):
            return plsc.bitcast(plsc.load_gather(sc_v, [phys(v)]), jnp.int32)

        srcs = [None, val_b, val_a, val_b]
        dsts = [val_b, val_a, val_b, val_a]
        hq = (hist0, hist1)
        NQ = 2
        SUB = CHUNK // NQ
        for p in range(PASSES):
            shift = 8 * p
            src_v = srcs[p]
            dst_v = dsts[p]

            def zero_body(j, _):
                for u in range(UNROLL):
                    for q in range(NQ):
                        hq[q][j * UNROLL + u, :] = zero_v
                return 0
            lax.fori_loop(0, RADIX // UNROLL, zero_body, 0)

            def hist_body(i, _):
                for q in range(NQ):
                    idxv = lane_c + (q * SUB + i)
                    v = idxv if src_v is None else plsc.load_gather(src_v, [idxv + lane])
                    d = _digit(key_of(v), shift)
                    plsc.addupdate_scatter(hq[q], [d, lane], ones)
                return 0
            lax.fori_loop(0, SUB, hist_body, 0)

            def cs_body(j, _):
                for u in range(2):
                    jj = j * 2 + u
                    t = hq[0][jj, :] + hq[1][jj, :]
                    hq[1][jj, :] = plsc.cumsum(t)
                return 0
            lax.fori_loop(0, RADIX // 2, cs_body, 0)

            def rb_body(jj, carry):
                rows_vec = lane + jj * L
                tot = plsc.load_gather(hq[1], [rows_vec, fifteen])
                cs2 = plsc.cumsum(tot)
                plsc.store_scatter(rowbase, [rows_vec], cs2 - tot + carry)
                return carry + jnp.sum(tot)
            lax.fori_loop(0, RADIX // L, rb_body, jnp.int32(0))

            def fin_body(j, _):
                for u in range(2):
                    jj = j * 2 + u
                    jb = jnp.broadcast_to(jj, (L,))
                    c0 = hq[0][jj, :]
                    incl = hq[1][jj, :]
                    shifted = plsc.load_gather(hq[1], [jb, lane_m1])
                    excl = jnp.where(lane > 0, shifted, 0)
                    rb = plsc.load_gather(rowbase, [jb])
                    base = excl + rb
                    hq[0][jj, :] = base
                    hq[1][jj, :] = base + c0
                return 0
            lax.fori_loop(0, RADIX // 2, fin_body, 0)

            def perm_body(i, _):
                for q in range(NQ):
                    idxv = lane_c + (q * SUB + i)
                    v = idxv if src_v is None else plsc.load_gather(src_v, [idxv + lane])
                    d = _digit(key_of(v), shift)
                    ofs = plsc.load_gather(hq[q], [d, lane])
                    plsc.store_scatter(dst_v, [phys(ofs)], v)
                    plsc.addupdate_scatter(hq[q], [d, lane], ones)
                return 0
            lax.fori_loop(0, SUB, perm_body, 0)

        def out_body(i, _):
            for u in range(UNROLL):
                e0 = (i * UNROLL + u) * L
                sl = pl.ds(e0, L)
                v = plsc.load_gather(val_a, [phys(e0 + lane)])
                k = key_of(v)
                vstage[sl] = plsc.bitcast(_desc_key(k), jnp.float32)
                gstage[sl] = v
            return 0
        lax.fori_loop(0, K // L // UNROLL, out_body, 0)

        pltpu.sync_copy(vstage, vals_hbm.at[pl.ds(b * K, K)])
        pltpu.sync_copy(gstage, idx_hbm.at[pl.ds(b * K, K)])

        def grow_body(i, _):
            for u in range(UNROLL):
                sl = pl.ds((i * UNROLL + u) * L, L)
                gstage[sl] = gstage[sl] + b * T
            return 0
        lax.fori_loop(0, K // L // UNROLL, grow_body, 0)
        pltpu.sync_copy(gstage, sh_grow.at[pl.ds(s * K, K)])

        def bidx_body(i, _):
            for u in range(UNROLL):
                gstage[pl.ds((i * UNROLL + u) * L, L)] = jnp.broadcast_to(b, (L,))
            return 0
        lax.fori_loop(0, K // L // UNROLL, bidx_body, 0)
        pltpu.sync_copy(gstage, bidx_hbm.at[pl.ds(b * K, K)])

    plsc.subcore_barrier()

    gbase = c * RPC + s * RPW

    bufs2 = (buf0, buf1)
    sems2 = (sem0, sem1)
    idxb = (idx0, idx1)

    def start(ch):
        pltpu.sync_copy(sh_grow.at[pl.ds(s * RPW + ch * GCH, GCH)], idxb[ch % 2])
        return pltpu.async_copy(
            hid_hbm.at[idxb[ch % 2]], bufs2[ch % 2], sems2[ch % 2])

    pending = start(0)
    for ch in range(NCH):
        nxt = start(ch + 1) if ch + 1 < NCH else None
        pending.wait()
        pltpu.sync_copy(bufs2[ch % 2], sel_hbm.at[pl.ds(gbase + ch * GCH, GCH)])
        pending = nxt


_fused_call = functools.partial(
    pl.kernel,
    out_type=(
        jax.ShapeDtypeStruct((B * K, D), jnp.float32),
        jax.ShapeDtypeStruct((B * K,), jnp.int32),
        jax.ShapeDtypeStruct((B * K,), jnp.int32),
        jax.ShapeDtypeStruct((B * K,), jnp.float32),
    ),
    mesh=plsc.VectorSubcoreMesh(core_axis_name="c", subcore_axis_name="s"),
    compiler_params=pltpu.CompilerParams(needs_layout_passes=False),
    scratch_types=[
        pltpu.VMEM((T + 16,), jnp.float32),
        pltpu.VMEM((T + 16,), jnp.int32),
        pltpu.VMEM((T + 16,), jnp.int32),
        pltpu.VMEM((RADIX, L), jnp.int32),
        pltpu.VMEM((RADIX, L), jnp.int32),
        pltpu.VMEM((RADIX,), jnp.int32),
        pltpu.VMEM((K,), jnp.float32),
        pltpu.VMEM((K,), jnp.int32),
        pltpu.VMEM((GCH,), jnp.int32),
        pltpu.VMEM((GCH,), jnp.int32),
        pltpu.VMEM((GCH, D), jnp.float32),
        pltpu.VMEM((GCH, D), jnp.float32),
        pltpu.VMEM_SHARED((RPC,), jnp.int32),
        pltpu.SemaphoreType.DMA,
        pltpu.SemaphoreType.DMA,
    ],
)(_body)


def kernel(scores, hidden_states):
    b, t, d = hidden_states.shape
    sel, bidx, idx, vals = _fused_call(
        scores.reshape(-1), hidden_states.reshape(b * t, d))
    return sel, bidx, idx, vals


# speedup vs baseline: 1.1216x; 1.0213x over previous
import functools

import jax
import jax.numpy as jnp
from jax import lax
from jax.experimental import pallas as pl
from jax.experimental.pallas import tpu as pltpu
from jax.experimental.pallas import tpu_sc as plsc

NC = 2
NS = 16
L = 16

B = 4
T = 4096
D = 2048
K = T // 2
RADIX = 256
PASSES = 4
CHUNK = T // L

RPC = 2 * K
RPW = RPC // NS
GCH = 8
NCH = RPW // GCH


def _digit(k_i32, shift):
    ku = plsc.bitcast(k_i32, jnp.uint32)
    du = jnp.bitwise_and(jnp.right_shift(ku, jnp.uint32(shift)), jnp.uint32(RADIX - 1))
    return plsc.bitcast(du, jnp.int32)


def _desc_key(bits_i32):
    sign = jnp.right_shift(bits_i32, 31)
    mask = jnp.bitwise_and(jnp.bitwise_not(sign), jnp.int32(0x7FFFFFFF))
    return jnp.bitwise_xor(bits_i32, mask)


def _body(scores_hbm, hid_hbm, sel_hbm, bidx_hbm, idx_hbm, vals_hbm,
          sc_v, key_a, key_b, val_a, val_b, hist, offs,
          gstage, idx_v, buf0, buf1, sh_grow, sem0, sem1):
    c = lax.axis_index("c")
    s = lax.axis_index("s")

    @pl.when(s < 2)
    def _():
        b = c * 2 + s
        pltpu.sync_copy(scores_hbm.at[pl.ds(b * T, T)], sc_v)

        lane = lax.iota(jnp.int32, L)
        lane_c = lane * CHUNK
        ones = jnp.broadcast_to(jnp.int32(1), (L,))

        def init_body(i, _):
            x = sc_v[pl.ds(i * L, L)]
            bits = plsc.bitcast(x, jnp.int32)
            key_a[pl.ds(i * L, L)] = _desc_key(bits)
            val_a[pl.ds(i * L, L)] = lane + i * L
            return 0
        lax.fori_loop(0, T // L, init_body, 0)

        bufs = [(key_a, val_a), (key_b, val_b)]
        for p in range(PASSES):
            shift = 8 * p
            src_k, src_v = bufs[p % 2]
            dst_k, dst_v = bufs[(p + 1) % 2]

            def zero_body(j, _):
                hist[j, :] = jnp.broadcast_to(jnp.int32(0), (L,))
                return 0
            lax.fori_loop(0, RADIX, zero_body, 0)

            def hist_body(i, _):
                idxv = lane_c + i
                k = plsc.load_gather(src_k, [idxv])
                d = _digit(k, shift)
                plsc.addupdate_scatter(hist, [d, lane], ones)
                return 0
            lax.fori_loop(0, CHUNK, hist_body, 0)

            def offs_body(dd, carry):
                row = hist[dd, :]
                cs = plsc.cumsum(row)
                offs[dd, :] = cs - row + carry
                return carry + jnp.sum(row)
            lax.fori_loop(0, RADIX, offs_body, jnp.int32(0))

            def perm_body(i, _):
                idxv = lane_c + i
                k = plsc.load_gather(src_k, [idxv])
                v = plsc.load_gather(src_v, [idxv])
                d = _digit(k, shift)
                ofs = plsc.load_gather(offs, [d, lane])
                plsc.store_scatter(dst_k, [ofs], k)
                plsc.store_scatter(dst_v, [ofs], v)
                plsc.addupdate_scatter(offs, [d, lane], ones)
                return 0
            lax.fori_loop(0, CHUNK, perm_body, 0)

        def out_body(i, _):
            k = key_a[pl.ds(i * L, L)]
            v = val_a[pl.ds(i * L, L)]
            sc_v[pl.ds(i * L, L)] = plsc.bitcast(_desc_key(k), jnp.float32)
            gstage[pl.ds(i * L, L)] = v + b * T
            return 0
        lax.fori_loop(0, K // L, out_body, 0)

        pltpu.sync_copy(sc_v.at[pl.ds(0, K)], vals_hbm.at[pl.ds(b * K, K)])
        pltpu.sync_copy(val_a.at[pl.ds(0, K)], idx_hbm.at[pl.ds(b * K, K)])
        pltpu.sync_copy(gstage, sh_grow.at[pl.ds(s * K, K)])

        def bidx_body(i, _):
            gstage[pl.ds(i * L, L)] = jnp.broadcast_to(b, (L,))
            return 0
        lax.fori_loop(0, K // L, bidx_body, 0)
        pltpu.sync_copy(gstage, bidx_hbm.at[pl.ds(b * K, K)])

    plsc.subcore_barrier()

    pltpu.sync_copy(sh_grow.at[pl.ds(s * RPW, RPW)], idx_v)
    gbase = c * RPC + s * RPW

    bufs2 = (buf0, buf1)
    sems2 = (sem0, sem1)

    def start(ch):
        return pltpu.async_copy(
            hid_hbm.at[idx_v.at[pl.ds(ch * GCH, GCH)]], bufs2[ch % 2], sems2[ch % 2])

    pending = start(0)
    for ch in range(NCH):
        nxt = start(ch + 1) if ch + 1 < NCH else None
        pending.wait()
        pltpu.sync_copy(bufs2[ch % 2], sel_hbm.at[pl.ds(gbase + ch * GCH, GCH)])
        pending = nxt


_fused_call = functools.partial(
    pl.kernel,
    out_type=(
        jax.ShapeDtypeStruct((B * K, D), jnp.float32),
        jax.ShapeDtypeStruct((B * K,), jnp.int32),
        jax.ShapeDtypeStruct((B * K,), jnp.int32),
        jax.ShapeDtypeStruct((B * K,), jnp.float32),
    ),
    mesh=plsc.VectorSubcoreMesh(core_axis_name="c", subcore_axis_name="s"),
    compiler_params=pltpu.CompilerParams(needs_layout_passes=False),
    scratch_types=[
        pltpu.VMEM((T,), jnp.float32),
        pltpu.VMEM((T,), jnp.int32),
        pltpu.VMEM((T,), jnp.int32),
        pltpu.VMEM((T,), jnp.int32),
        pltpu.VMEM((T,), jnp.int32),
        pltpu.VMEM((RADIX, L), jnp.int32),
        pltpu.VMEM((RADIX, L), jnp.int32),
        pltpu.VMEM((K,), jnp.int32),
        pltpu.VMEM((RPW,), jnp.int32),
        pltpu.VMEM((GCH, D), jnp.float32),
        pltpu.VMEM((GCH, D), jnp.float32),
        pltpu.VMEM_SHARED((RPC,), jnp.int32),
        pltpu.SemaphoreType.DMA,
        pltpu.SemaphoreType.DMA,
    ],
)(_body)


def kernel(scores, hidden_states):
    b, t, d = hidden_states.shape
    sel, bidx, idx, vals = _fused_call(
        scores.reshape(-1), hidden_states.reshape(b * t, d))
    return sel, bidx, idx, vals


# speedup vs baseline: 1.2617x; 1.1249x over previous
import functools

import jax
import jax.numpy as jnp
from jax import lax
from jax.experimental import pallas as pl
from jax.experimental.pallas import tpu as pltpu
from jax.experimental.pallas import tpu_sc as plsc

NC = 2
NS = 16
L = 16

B = 4
T = 4096
D = 2048
K = T // 2
RADIX = 256
PASSES = 4
CHUNK = T // L

RPC = 2 * K
RPW = RPC // NS
GCH = 8
NCH = RPW // GCH


def _digit(k_i32, shift):
    ku = plsc.bitcast(k_i32, jnp.uint32)
    du = jnp.bitwise_and(jnp.right_shift(ku, jnp.uint32(shift)), jnp.uint32(RADIX - 1))
    return plsc.bitcast(du, jnp.int32)


def _desc_key(bits_i32):
    sign = jnp.right_shift(bits_i32, 31)
    mask = jnp.bitwise_and(jnp.bitwise_not(sign), jnp.int32(0x7FFFFFFF))
    return jnp.bitwise_xor(bits_i32, mask)


def _body(scores_hbm, hid_hbm, sel_hbm, bidx_hbm, idx_hbm, vals_hbm,
          sc_v, key_a, key_b, val_a, val_b, hist, offs,
          gstage, idx_v, buf0, buf1, sh_grow, sem0, sem1):
    c = lax.axis_index("c")
    s = lax.axis_index("s")

    @pl.when(s < 2)
    def _():
        b = c * 2 + s
        pltpu.sync_copy(scores_hbm.at[pl.ds(b * T, T)], sc_v)

        lane = lax.iota(jnp.int32, L)
        lane_c = lane * CHUNK
        ones = jnp.broadcast_to(jnp.int32(1), (L,))

        def phys(e):
            return e + jnp.right_shift(e, 8)

        def init_body(i, _):
            x = sc_v[pl.ds(i * L, L)]
            bits = plsc.bitcast(x, jnp.int32)
            e = lane + i * L
            pe = phys(e)
            plsc.store_scatter(key_a, [pe], _desc_key(bits))
            plsc.store_scatter(val_a, [pe], e)
            return 0
        lax.fori_loop(0, T // L, init_body, 0)

        bufs = [(key_a, val_a), (key_b, val_b)]
        for p in range(PASSES):
            shift = 8 * p
            src_k, src_v = bufs[p % 2]
            dst_k, dst_v = bufs[(p + 1) % 2]

            def zero_body(j, _):
                hist[j, :] = jnp.broadcast_to(jnp.int32(0), (L,))
                return 0
            lax.fori_loop(0, RADIX, zero_body, 0)

            def hist_body(i, _):
                idxv = lane_c + i + lane
                k = plsc.load_gather(src_k, [idxv])
                d = _digit(k, shift)
                plsc.addupdate_scatter(hist, [d, lane], ones)
                return 0
            lax.fori_loop(0, CHUNK, hist_body, 0)

            def offs_body(dd, carry):
                row = hist[dd, :]
                cs = plsc.cumsum(row)
                offs[dd, :] = cs - row + carry
                return carry + jnp.sum(row)
            lax.fori_loop(0, RADIX, offs_body, jnp.int32(0))

            def perm_body(i, _):
                idxv = lane_c + i + lane
                k = plsc.load_gather(src_k, [idxv])
                v = plsc.load_gather(src_v, [idxv])
                d = _digit(k, shift)
                ofs = plsc.load_gather(offs, [d, lane])
                po = phys(ofs)
                plsc.store_scatter(dst_k, [po], k)
                plsc.store_scatter(dst_v, [po], v)
                plsc.addupdate_scatter(offs, [d, lane], ones)
                return 0
            lax.fori_loop(0, CHUNK, perm_body, 0)

        def out_body(i, _):
            pe = phys(lane + i * L)
            k = plsc.load_gather(key_a, [pe])
            v = plsc.load_gather(val_a, [pe])
            sc_v[pl.ds(i * L, L)] = plsc.bitcast(_desc_key(k), jnp.float32)
            gstage[pl.ds(i * L, L)] = v
            return 0
        lax.fori_loop(0, K // L, out_body, 0)

        pltpu.sync_copy(sc_v.at[pl.ds(0, K)], vals_hbm.at[pl.ds(b * K, K)])
        pltpu.sync_copy(gstage, idx_hbm.at[pl.ds(b * K, K)])

        def grow_body(i, _):
            gstage[pl.ds(i * L, L)] = gstage[pl.ds(i * L, L)] + b * T
            return 0
        lax.fori_loop(0, K // L, grow_body, 0)
        pltpu.sync_copy(gstage, sh_grow.at[pl.ds(s * K, K)])

        def bidx_body(i, _):
            gstage[pl.ds(i * L, L)] = jnp.broadcast_to(b, (L,))
            return 0
        lax.fori_loop(0, K // L, bidx_body, 0)
        pltpu.sync_copy(gstage, bidx_hbm.at[pl.ds(b * K, K)])

    plsc.subcore_barrier()

    pltpu.sync_copy(sh_grow.at[pl.ds(s * RPW, RPW)], idx_v)
    gbase = c * RPC + s * RPW

    bufs2 = (buf0, buf1)
    sems2 = (sem0, sem1)

    def start(ch):
        return pltpu.async_copy(
            hid_hbm.at[idx_v.at[pl.ds(ch * GCH, GCH)]], bufs2[ch % 2], sems2[ch % 2])

    pending = start(0)
    for ch in range(NCH):
        nxt = start(ch + 1) if ch + 1 < NCH else None
        pending.wait()
        pltpu.sync_copy(bufs2[ch % 2], sel_hbm.at[pl.ds(gbase + ch * GCH, GCH)])
        pending = nxt


_fused_call = functools.partial(
    pl.kernel,
    out_type=(
        jax.ShapeDtypeStruct((B * K, D), jnp.float32),
        jax.ShapeDtypeStruct((B * K,), jnp.int32),
        jax.ShapeDtypeStruct((B * K,), jnp.int32),
        jax.ShapeDtypeStruct((B * K,), jnp.float32),
    ),
    mesh=plsc.VectorSubcoreMesh(core_axis_name="c", subcore_axis_name="s"),
    compiler_params=pltpu.CompilerParams(needs_layout_passes=False),
    scratch_types=[
        pltpu.VMEM((T,), jnp.float32),
        pltpu.VMEM((T + 16,), jnp.int32),
        pltpu.VMEM((T + 16,), jnp.int32),
        pltpu.VMEM((T + 16,), jnp.int32),
        pltpu.VMEM((T + 16,), jnp.int32),
        pltpu.VMEM((RADIX, L), jnp.int32),
        pltpu.VMEM((RADIX, L), jnp.int32),
        pltpu.VMEM((K,), jnp.int32),
        pltpu.VMEM((RPW,), jnp.int32),
        pltpu.VMEM((GCH, D), jnp.float32),
        pltpu.VMEM((GCH, D), jnp.float32),
        pltpu.VMEM_SHARED((RPC,), jnp.int32),
        pltpu.SemaphoreType.DMA,
        pltpu.SemaphoreType.DMA,
    ],
)(_body)


def kernel(scores, hidden_states):
    b, t, d = hidden_states.shape
    sel, bidx, idx, vals = _fused_call(
        scores.reshape(-1), hidden_states.reshape(b * t, d))
    return sel, bidx, idx, vals
